# baseline (speedup 1.0000x reference)
import functools

import jax
import jax.numpy as jnp
from jax import lax
from jax.experimental import pallas as pl
from jax.experimental.pallas import tpu as pltpu
from jax.experimental.pallas import tpu_sc as plsc

N_NODES = 10000
D = 128
S = 32
C = 16
B = 10000
NW = 32
NP = 10240
PER_TILE = NP // NW
CPN = 1
NBUF = 2
NCH = PER_TILE // CPN
IDX_CHUNK = 80


def _sc_neigh_sums():
    mesh = plsc.VectorSubcoreMesh(core_axis_name="c", subcore_axis_name="s")

    @functools.partial(
        pl.kernel,
        out_type=jax.ShapeDtypeStruct((NP, D), jnp.float32),
        mesh=mesh,
        scratch_types=(
            pltpu.VMEM((PER_TILE, D), jnp.int32),
            pltpu.VMEM((PER_TILE, D), jnp.float32),
            pltpu.VMEM((CPN * S, D), jnp.float32),
            pltpu.VMEM((CPN * S, D), jnp.float32),
            pltpu.VMEM((CPN * S, D), jnp.float32),
            pltpu.VMEM((CPN * S, D), jnp.float32),
            pltpu.SemaphoreType.DMA,
            pltpu.SemaphoreType.DMA,
            pltpu.SemaphoreType.DMA,
            pltpu.SemaphoreType.DMA,
        ),
    )
    def k(feats_hbm, neigh_hbm, nsum_out,
          nbf_v, nsum_v, buf0, buf1, buf2, buf3,
          sem0, sem1, sem2, sem3):
        wid = lax.axis_index("c") * 16 + lax.axis_index("s")
        base = wid * PER_TILE

        pltpu.sync_copy(neigh_hbm.at[pl.ds(base, PER_TILE)], nbf_v)

        bufs = (buf0, buf1, buf2, buf3)[:NBUF]
        sems = (sem0, sem1, sem2, sem3)[:NBUF]

        def idx_ref(ch):
            return nbf_v.at[ch, pl.ds(0, S)]

        for k0 in range(NBUF):
            pltpu.async_copy(feats_hbm.at[idx_ref(k0)], bufs[k0], sems[k0])

        @pl.loop(0, NCH, step=NBUF)
        def _(c0):
            for kb in range(NBUF):
                ch = c0 + kb
                buf = bufs[kb]
                sem = sems[kb]
                pltpu.make_async_copy(feats_hbm.at[idx_ref(ch)], buf,
                                      sem).wait()
                for j in range(CPN):
                    for cg in range(D // 16):
                        sl = pl.ds(cg * 16, 16)
                        vals = [buf[j * S + s, sl] for s in range(S)]
                        if True:
                            nsum_v[ch * CPN + j, sl] = vals[0]
                            continue
                        while len(vals) > 1:
                            vals = [vals[t] + vals[t + 1]
                                    for t in range(0, len(vals) - 1, 2)] + (
                                        [vals[-1]] if len(vals) % 2 else [])
                        nsum_v[ch * CPN + j, sl] = vals[0]

                @pl.when(ch + NBUF < NCH)
                def _():
                    pltpu.async_copy(feats_hbm.at[idx_ref(ch + NBUF)], buf,
                                     sem)

        pltpu.sync_copy(nsum_v, nsum_out.at[pl.ds(base, PER_TILE)])

    return k


def _sc_row_gather():
    mesh = plsc.VectorSubcoreMesh(core_axis_name="c", subcore_axis_name="s")

    @functools.partial(
        pl.kernel,
        out_type=jax.ShapeDtypeStruct((NP, D), jnp.float32),
        mesh=mesh,
        scratch_types=(
            pltpu.VMEM((PER_TILE,), jnp.int32),
            pltpu.VMEM((PER_TILE, D), jnp.float32),
            pltpu.SemaphoreType.DMA,
        ),
    )
    def k(nodes_hbm, scores_hbm, out_hbm, nodes_v, rows_v, sem):
        wid = lax.axis_index("s") * 2 + lax.axis_index("c")
        base = wid * PER_TILE
        n_chunks = PER_TILE // IDX_CHUNK

        pltpu.sync_copy(nodes_hbm.at[pl.ds(base, PER_TILE)], nodes_v)
        for j in range(n_chunks):
            idx = nodes_v.at[pl.ds(j * IDX_CHUNK, IDX_CHUNK)]
            pltpu.async_copy(scores_hbm.at[idx],
                             rows_v.at[pl.ds(j * IDX_CHUNK, IDX_CHUNK)], sem)
        for j in range(n_chunks):
            pltpu.make_async_copy(
                scores_hbm.at[nodes_v.at[pl.ds(j * IDX_CHUNK, IDX_CHUNK)]],
                rows_v.at[pl.ds(j * IDX_CHUNK, IDX_CHUNK)], sem).wait()
        pltpu.sync_copy(rows_v, out_hbm.at[pl.ds(base, PER_TILE)])

    return k


TC_BLK = 400


def _tc_dense(xs, xn, ws_t, wn_t, wc_pad):
    def body(xs_ref, xn_ref, ws_ref, wn_ref, wc_ref, out_ref):
        h = jnp.dot(xs_ref[...], ws_ref[...], preferred_element_type=jnp.float32)
        h += jnp.dot(xn_ref[...], wn_ref[...], preferred_element_type=jnp.float32)
        h = jnp.maximum(h, 0.0)
        out_ref[...] = jnp.dot(h, wc_ref[...], preferred_element_type=jnp.float32)

    grid = N_NODES // TC_BLK
    return pl.pallas_call(
        body,
        grid=(grid,),
        in_specs=[
            pl.BlockSpec((TC_BLK, D), lambda i: (i, 0)),
            pl.BlockSpec((TC_BLK, D), lambda i: (i, 0)),
            pl.BlockSpec((D, D), lambda i: (0, 0)),
            pl.BlockSpec((D, D), lambda i: (0, 0)),
            pl.BlockSpec((D, D), lambda i: (0, 0)),
        ],
        out_specs=pl.BlockSpec((TC_BLK, D), lambda i: (i, 0)),
        out_shape=jax.ShapeDtypeStruct((N_NODES, D), jnp.float32),
    )(xs, xn, ws_t, wn_t, wc_pad)


def kernel(nodes, features, neigh_idx, W_enc, W_cls):
    neigh_p = jnp.pad(neigh_idx, ((0, NP - N_NODES), (0, D - S)))
    nsum = _sc_neigh_sums()(features, neigh_p)
    return nsum[:B, :C]
    ws_t = W_enc[:, :D].T
    wn_t = W_enc[:, D:].T * (1.0 / S)
    wc_pad = jnp.pad(W_cls.T, ((0, 0), (0, D - C)))
    scores_all = _tc_dense(features, nsum, ws_t, wn_t, wc_pad)
    nodes_p = jnp.pad(nodes.astype(jnp.int32), (0, NP - B))
    outp = _sc_row_gather()(nodes_p, scores_all)
    return outp[:B, :C]


# speedup vs baseline: 2.1633x; 2.1633x over previous
import functools

import jax
import jax.numpy as jnp
from jax import lax
from jax.experimental import pallas as pl
from jax.experimental.pallas import tpu as pltpu
from jax.experimental.pallas import tpu_sc as plsc

N_NODES = 10000
D = 128
S = 32
C = 16
B = 10000
NW = 32
BP = 10240
PER_TILE = BP // NW
IDX_CHUNK = 80
NBUF = 2


def _sc_gather_mean():
    mesh = plsc.VectorSubcoreMesh(core_axis_name="c", subcore_axis_name="s")

    @functools.partial(
        pl.kernel,
        out_type=(
            jax.ShapeDtypeStruct((BP, D), jnp.float32),
            jax.ShapeDtypeStruct((BP, D), jnp.float32),
        ),
        mesh=mesh,
        scratch_types=(
            pltpu.VMEM((PER_TILE,), jnp.int32),
            pltpu.VMEM((PER_TILE, D), jnp.int32),
            pltpu.VMEM((PER_TILE, D), jnp.float32),
            tuple(pltpu.VMEM((S, D), jnp.float32) for _ in range(NBUF)),
            pltpu.VMEM((IDX_CHUNK, D), jnp.float32),
            pltpu.VMEM((IDX_CHUNK, D), jnp.float32),
            pltpu.SemaphoreType.DMA,
            pltpu.SemaphoreType.DMA,
            tuple(pltpu.SemaphoreType.DMA for _ in range(NBUF)),
        ),
    )
    def k(nodes_hbm, feats_hbm, neigh_hbm, self_out, nsum_out,
          nodes_v, nb_v, nsum_v, bufs, sbuf0, sbuf1,
          sem_nb, sem_s, sems):
        wid = lax.axis_index("s") * 2 + lax.axis_index("c")
        base = wid * PER_TILE
        n_chunks = PER_TILE // IDX_CHUNK

        pltpu.sync_copy(nodes_hbm.at[pl.ds(base, PER_TILE)], nodes_v)

        for j in range(n_chunks):
            idx = nodes_v.at[pl.ds(j * IDX_CHUNK, IDX_CHUNK)]
            pltpu.async_copy(neigh_hbm.at[idx],
                             nb_v.at[pl.ds(j * IDX_CHUNK, IDX_CHUNK)], sem_nb)
        for j in range(n_chunks):
            pltpu.make_async_copy(
                neigh_hbm.at[nodes_v.at[pl.ds(j * IDX_CHUNK, IDX_CHUNK)]],
                nb_v.at[pl.ds(j * IDX_CHUNK, IDX_CHUNK)], sem_nb).wait()

        for k0 in range(NBUF):
            pltpu.async_copy(feats_hbm.at[nb_v.at[k0, pl.ds(0, S)]],
                             bufs[k0], sems[k0])

        @pl.loop(0, PER_TILE, step=NBUF)
        def _(i0):
            for b in range(NBUF):
                i = i0 + b
                buf = bufs[b]
                sem = sems[b]
                pltpu.make_async_copy(
                    feats_hbm.at[nb_v.at[i, pl.ds(0, S)]], buf, sem).wait()
                for c in range(D // 16):
                    sl = pl.ds(c * 16, 16)
                    vals = [buf[s, sl] for s in range(S)]
                    while len(vals) > 1:
                        vals = [vals[t] + vals[t + 1]
                                for t in range(0, len(vals) - 1, 2)] + (
                                    [vals[-1]] if len(vals) % 2 else [])
                    nsum_v[i, sl] = vals[0]

                @pl.when(i < PER_TILE - NBUF)
                def _():
                    pltpu.async_copy(
                        feats_hbm.at[nb_v.at[i + NBUF, pl.ds(0, S)]], buf,
                        sem)

        sbufs = (sbuf0, sbuf1)
        pltpu.async_copy(feats_hbm.at[nodes_v.at[pl.ds(0, IDX_CHUNK)]],
                         sbuf0, sem_s)
        for j in range(n_chunks):
            idx = nodes_v.at[pl.ds(j * IDX_CHUNK, IDX_CHUNK)]
            sb = sbufs[j % 2]
            pltpu.make_async_copy(feats_hbm.at[idx], sb, sem_s).wait()
            if j + 1 < n_chunks:
                nidx = nodes_v.at[pl.ds((j + 1) * IDX_CHUNK, IDX_CHUNK)]
                pltpu.async_copy(feats_hbm.at[nidx], sbufs[(j + 1) % 2], sem_s)
            pltpu.sync_copy(sb, self_out.at[pl.ds(base + j * IDX_CHUNK,
                                                  IDX_CHUNK)])

        pltpu.sync_copy(nsum_v, nsum_out.at[pl.ds(base, PER_TILE)])

    return k


TC_BLK = 1024


def _tc_dense(xs, xn, ws_t, wn_t, wc_t):
    def body(xs_ref, xn_ref, ws_ref, wn_ref, wc_ref, out_ref):
        h = jnp.dot(xs_ref[...], ws_ref[...], preferred_element_type=jnp.float32)
        h += jnp.dot(xn_ref[...], wn_ref[...], preferred_element_type=jnp.float32)
        h = jnp.maximum(h, 0.0)
        out_ref[...] = jnp.dot(h, wc_ref[...], preferred_element_type=jnp.float32)

    grid = BP // TC_BLK
    return pl.pallas_call(
        body,
        grid=(grid,),
        in_specs=[
            pl.BlockSpec((TC_BLK, D), lambda i: (i, 0)),
            pl.BlockSpec((TC_BLK, D), lambda i: (i, 0)),
            pl.BlockSpec((D, D), lambda i: (0, 0)),
            pl.BlockSpec((D, D), lambda i: (0, 0)),
            pl.BlockSpec((D, C), lambda i: (0, 0)),
        ],
        out_specs=pl.BlockSpec((TC_BLK, C), lambda i: (i, 0)),
        out_shape=jax.ShapeDtypeStruct((BP, C), jnp.float32),
    )(xs, xn, ws_t, wn_t, wc_t)


def kernel(nodes, features, neigh_idx, W_enc, W_cls):
    nodes_p = jnp.pad(nodes.astype(jnp.int32), (0, BP - B))
    neigh_p = jnp.pad(neigh_idx, ((0, 0), (0, D - S)))
    self_f, nsum = _sc_gather_mean()(nodes_p, features, neigh_p)
    ws_t = W_enc[:, :D].T
    wn_t = W_enc[:, D:].T * (1.0 / S)
    wc_t = W_cls.T
    scores = _tc_dense(self_f, nsum, ws_t, wn_t, wc_t)
    return scores[:B]


# speedup vs baseline: 3.0153x; 1.3939x over previous
import functools

import jax
import jax.numpy as jnp
from jax import lax
from jax.experimental import pallas as pl
from jax.experimental.pallas import tpu as pltpu
from jax.experimental.pallas import tpu_sc as plsc

N_NODES = 10000
D = 128
S = 32
C = 16
B = 10000
NW = 32
BP = 10240
PER_TILE = BP // NW
IDX_CHUNK = 80
NBUF = 4


def _sc_gather_mean():
    mesh = plsc.VectorSubcoreMesh(core_axis_name="c", subcore_axis_name="s")

    @functools.partial(
        pl.kernel,
        out_type=(
            jax.ShapeDtypeStruct((BP, D), jnp.float32),
            jax.ShapeDtypeStruct((BP, D), jnp.float32),
        ),
        mesh=mesh,
        scratch_types=(
            pltpu.VMEM((PER_TILE,), jnp.int32),
            pltpu.VMEM((PER_TILE, D), jnp.int32),
            pltpu.VMEM((PER_TILE, D), jnp.float32),
            tuple(pltpu.VMEM((S, D), jnp.float32) for _ in range(NBUF)),
            pltpu.VMEM((IDX_CHUNK, D), jnp.float32),
            pltpu.VMEM((IDX_CHUNK, D), jnp.float32),
            pltpu.SemaphoreType.DMA,
            pltpu.SemaphoreType.DMA,
            tuple(pltpu.SemaphoreType.DMA for _ in range(NBUF)),
        ),
    )
    def k(nodes_hbm, feats_hbm, neigh_hbm, self_out, nsum_out,
          nodes_v, nb_v, nsum_v, bufs, sbuf0, sbuf1,
          sem_nb, sem_s, sems):
        wid = lax.axis_index("s") * 2 + lax.axis_index("c")
        base = wid * PER_TILE
        n_chunks = PER_TILE // IDX_CHUNK

        pltpu.sync_copy(nodes_hbm.at[pl.ds(base, PER_TILE)], nodes_v)

        for j in range(n_chunks):
            idx = nodes_v.at[pl.ds(j * IDX_CHUNK, IDX_CHUNK)]
            pltpu.async_copy(neigh_hbm.at[idx],
                             nb_v.at[pl.ds(j * IDX_CHUNK, IDX_CHUNK)], sem_nb)
        for j in range(n_chunks):
            pltpu.make_async_copy(
                neigh_hbm.at[nodes_v.at[pl.ds(j * IDX_CHUNK, IDX_CHUNK)]],
                nb_v.at[pl.ds(j * IDX_CHUNK, IDX_CHUNK)], sem_nb).wait()

        for k0 in range(NBUF):
            pltpu.async_copy(feats_hbm.at[nb_v.at[k0, pl.ds(0, S)]],
                             bufs[k0], sems[k0])

        @pl.loop(0, PER_TILE, step=NBUF)
        def _(i0):
            for b in range(NBUF):
                i = i0 + b
                buf = bufs[b]
                sem = sems[b]
                pltpu.make_async_copy(
                    feats_hbm.at[nb_v.at[i, pl.ds(0, S)]], buf, sem).wait()
                for c in range(D // 16):
                    sl = pl.ds(c * 16, 16)
                    vals = [buf[s, sl] for s in range(S)]
                    while len(vals) > 1:
                        vals = [vals[t] + vals[t + 1]
                                for t in range(0, len(vals) - 1, 2)] + (
                                    [vals[-1]] if len(vals) % 2 else [])
                    nsum_v[i, sl] = vals[0]

                @pl.when(i < PER_TILE - NBUF)
                def _():
                    pltpu.async_copy(
                        feats_hbm.at[nb_v.at[i + NBUF, pl.ds(0, S)]], buf,
                        sem)

        sbufs = (sbuf0, sbuf1)
        pltpu.async_copy(feats_hbm.at[nodes_v.at[pl.ds(0, IDX_CHUNK)]],
                         sbuf0, sem_s)
        for j in range(n_chunks):
            idx = nodes_v.at[pl.ds(j * IDX_CHUNK, IDX_CHUNK)]
            sb = sbufs[j % 2]
            pltpu.make_async_copy(feats_hbm.at[idx], sb, sem_s).wait()
            if j + 1 < n_chunks:
                nidx = nodes_v.at[pl.ds((j + 1) * IDX_CHUNK, IDX_CHUNK)]
                pltpu.async_copy(feats_hbm.at[nidx], sbufs[(j + 1) % 2], sem_s)
            pltpu.sync_copy(sb, self_out.at[pl.ds(base + j * IDX_CHUNK,
                                                  IDX_CHUNK)])

        pltpu.sync_copy(nsum_v, nsum_out.at[pl.ds(base, PER_TILE)])

    return k


TC_BLK = 1024


def _tc_dense(xs, xn, ws_t, wn_t, wc_t):
    def body(xs_ref, xn_ref, ws_ref, wn_ref, wc_ref, out_ref):
        h = jnp.dot(xs_ref[...], ws_ref[...], preferred_element_type=jnp.float32)
        h += jnp.dot(xn_ref[...], wn_ref[...], preferred_element_type=jnp.float32)
        h = jnp.maximum(h, 0.0)
        out_ref[...] = jnp.dot(h, wc_ref[...], preferred_element_type=jnp.float32)

    grid = BP // TC_BLK
    return pl.pallas_call(
        body,
        grid=(grid,),
        in_specs=[
            pl.BlockSpec((TC_BLK, D), lambda i: (i, 0)),
            pl.BlockSpec((TC_BLK, D), lambda i: (i, 0)),
            pl.BlockSpec((D, D), lambda i: (0, 0)),
            pl.BlockSpec((D, D), lambda i: (0, 0)),
            pl.BlockSpec((D, C), lambda i: (0, 0)),
        ],
        out_specs=pl.BlockSpec((TC_BLK, C), lambda i: (i, 0)),
        out_shape=jax.ShapeDtypeStruct((BP, C), jnp.float32),
    )(xs, xn, ws_t, wn_t, wc_t)


def kernel(nodes, features, neigh_idx, W_enc, W_cls):
    nodes_p = jnp.pad(nodes.astype(jnp.int32), (0, BP - B))
    neigh_p = jnp.pad(neigh_idx, ((0, 0), (0, D - S)))
    self_f, nsum = _sc_gather_mean()(nodes_p, features, neigh_p)
    ws_t = W_enc[:, :D].T
    wn_t = W_enc[:, D:].T * (1.0 / S)
    wc_t = W_cls.T
    scores = _tc_dense(self_f, nsum, ws_t, wn_t, wc_t)
    return scores[:B]
